# baseline (speedup 1.0000x reference)
import functools

import numpy as np
import jax
import jax.numpy as jnp
from jax import lax
from jax.experimental import pallas as pl
from jax.experimental.pallas import tpu as pltpu
from jax.experimental.pallas import tpu_sc as plsc

LN_EPS = 1e-3
L = 16


def _posenc_np(max_len, d):
    pos = np.arange(max_len)[:, None]
    i = np.arange(d)[None, :]
    ang = pos * (1.0 / np.power(10000, 2 * (i // 2) / np.float32(d)))
    ang[:, 0::2] = np.sin(ang[:, 0::2])
    ang[:, 1::2] = np.cos(ang[:, 1::2])
    return ang.astype(np.float32)


def _rsqrt_vec(v):
    i = lax.bitcast_convert_type(v, jnp.int32)
    i = jnp.int32(0x5F3759DF) - lax.shift_right_logical(i, jnp.int32(1))
    y = lax.bitcast_convert_type(i, jnp.float32)
    for _ in range(2):
        y = y * (1.5 - 0.5 * v * y * y)
    return y


@functools.lru_cache(maxsize=None)
def _build(NT, E, S, CHUNK, UNROLL):
    info = plsc.get_sparse_core_info()
    NC, NS = info.num_cores, info.num_subcores
    NW = NC * NS
    TPW = NT // NW
    NCHUNK = TPW // CHUNK
    KD = E // L
    mesh = plsc.VectorSubcoreMesh(core_axis_name="c", subcore_axis_name="s")

    @functools.partial(
        pl.kernel,
        mesh=mesh,
        compiler_params=pltpu.CompilerParams(use_tc_tiling_on_sc=False),
        out_type=jax.ShapeDtypeStruct((NT, E), jnp.float32),
        scratch_types=[
            pltpu.VMEM((CHUNK,), jnp.int32),
            pltpu.VMEM((CHUNK,), jnp.float32),
            pltpu.VMEM((CHUNK, E), jnp.float32),
            pltpu.VMEM((S, E), jnp.float32),
            pltpu.VMEM((2, E), jnp.float32),
            pltpu.SemaphoreType.DMA,
        ],
    )
    def k(ids_hbm, ttf_hbm, table_hbm, pos_hbm, gb_hbm, out_hbm,
          idx_v, ttf_v, rows_v, pos_v, gb_v, sem):
        cid = lax.axis_index("c")
        sid = lax.axis_index("s")
        wid = sid * NC + cid
        t_base = wid * TPW

        pltpu.sync_copy(pos_hbm, pos_v)
        pltpu.sync_copy(gb_hbm, gb_v)
        gs = [gb_v[0, pl.ds(kk * L, L)] for kk in range(KD)]
        bs = [gb_v[1, pl.ds(kk * L, L)] for kk in range(KD)]

        lanes = lax.iota(jnp.int32, L)
        perms = [lanes ^ (1 << p) for p in range(4)]

        dnums = lax.GatherDimensionNumbers(
            offset_dims=(), collapsed_slice_dims=(0,), start_index_map=(0,))

        def hsum(v):
            for p in perms:
                v = v + lax.gather(v, p[:, None], dnums, (1,),
                                   mode=lax.GatherScatterMode.PROMISE_IN_BOUNDS)
            return v

        def do_token(t0, ii, tt):
            t = t0 + ii
            spos = lax.rem(t, S)
            xs = [rows_v[ii, pl.ds(kk * L, L)] + pos_v[spos, pl.ds(kk * L, L)] + tt
                  for kk in range(KD)]
            sv = (xs[0] + xs[1]) + (xs[2] + xs[3])
            s2v = (xs[0] * xs[0] + xs[1] * xs[1]) + (xs[2] * xs[2] + xs[3] * xs[3])
            mean = hsum(sv) * (1.0 / E)
            var = hsum(s2v) * (1.0 / E) - mean * mean
            rstd = _rsqrt_vec(var + LN_EPS)
            for kk in range(KD):
                rows_v[ii, pl.ds(kk * L, L)] = (xs[kk] - mean) * rstd * gs[kk] + bs[kk]

        def chunk_body(ci, carry):
            t0 = t_base + ci * CHUNK
            pltpu.sync_copy(ids_hbm.at[pl.ds(t0, CHUNK)], idx_v)
            pltpu.sync_copy(ttf_hbm.at[pl.ds(t0, CHUNK)], ttf_v)
            pltpu.async_copy(table_hbm.at[idx_v], rows_v, sem).wait()

            def grp_body(g, c2):
                base = g * L
                ttv = ttf_v[pl.ds(base, L)]
                for u in range(L):
                    do_token(t0, base + u, ttv[u])
                return c2

            lax.fori_loop(0, CHUNK // L, grp_body, 0)
            pltpu.sync_copy(rows_v, out_hbm.at[pl.ds(t0, CHUNK)])
            return carry

        lax.fori_loop(0, NCHUNK, chunk_body, 0)

    return k


def kernel(input_ids, token_type_ids, table, gamma, beta):
    B, S = input_ids.shape
    V, E = table.shape
    NT = B * S
    ids = input_ids.reshape(NT).astype(jnp.int32)
    ttf = token_type_ids.reshape(NT).astype(jnp.float32)
    pos = jnp.asarray(_posenc_np(S, E))
    gb = jnp.stack([gamma, beta])
    out = _build(NT, E, S, 256, 2)(ids, ttf, table, pos, gb)
    return out.reshape(B, S, E)
